# baseline (speedup 1.0000x reference)
import functools

import jax
import jax.numpy as jnp
from jax import lax
from jax.experimental import pallas as pl
from jax.experimental.pallas import tpu as pltpu
from jax.experimental.pallas import tpu_sc as plsc

NC = 2
NS = 16
CHUNK = 128


def _make_sc_agg(n_pad, d, cps, bpc):
    rps = n_pad // NS

    def body(x_hbm, src_hbm, dst_hbm, zrow_hbm, agg_out, deg_out,
             srcv, dstv, rows0, rows1, onesv, degv, agg_sh, deg_sh,
             semg0, semg1, semd):
        c = lax.axis_index("c")
        s = lax.axis_index("s")
        is_agg = c == 0

        @pl.when(is_agg)
        def _():
            pltpu.sync_copy(zrow_hbm.at[pl.ds(s * rps, rps)],
                            agg_sh.at[pl.ds(s * rps, rps)])

        for i in range(rps // 16):
            degv[pl.ds(i * 16, 16)] = jnp.zeros((16,), jnp.float32)
        pltpu.sync_copy(degv, deg_sh.at[pl.ds(s * rps, rps)])
        for i in range(CHUNK // 16):
            onesv[pl.ds(i * 16, 16)] = jnp.ones((16,), jnp.float32)
        plsc.subcore_barrier()

        bufs = (rows0, rows1)
        semgs = (semg0, semg1)

        for k in range(cps // bpc):
            base = s * cps + k * bpc
            pltpu.sync_copy(dst_hbm.at[pl.ds(base, bpc)], dstv)

            @pl.when(is_agg)
            def _():
                pltpu.sync_copy(src_hbm.at[pl.ds(base, bpc)], srcv)
                pltpu.async_copy(x_hbm.at[srcv.at[0]], rows0, semg0)
                pltpu.async_copy(x_hbm.at[srcv.at[1]], rows1, semg1)

                def step(i, carry):
                    for b in range(2):
                        j = i * 2 + b
                        rows = bufs[b]
                        pltpu.make_async_copy(x_hbm.at[srcv.at[j]], rows,
                                              semgs[b]).wait()
                        pltpu.sync_copy(rows, agg_sh.at[dstv.at[j]],
                                        add=True)

                        @pl.when(j + 2 < bpc)
                        def _():
                            pltpu.async_copy(x_hbm.at[srcv.at[j + 2]],
                                             rows, semgs[b])
                    return carry

                lax.fori_loop(0, bpc // 2, step, 0)

            @pl.when(jnp.logical_not(is_agg))
            def _():
                def fire(j, carry):
                    pltpu.async_copy(onesv, deg_sh.at[dstv.at[j]], semd,
                                     add=True)
                    return carry

                lax.fori_loop(0, bpc, fire, 0)

                def drain(j, carry):
                    pltpu.make_async_copy(onesv, deg_sh.at[dstv.at[j]],
                                          semd).wait()
                    return carry

                lax.fori_loop(0, bpc, drain, 0)

        plsc.subcore_barrier()

        @pl.when(is_agg)
        def _():
            pltpu.sync_copy(agg_sh.at[pl.ds(s * rps, rps)],
                            agg_out.at[pl.ds(s * rps, rps)])

        @pl.when(jnp.logical_not(is_agg))
        def _():
            pltpu.sync_copy(deg_sh.at[pl.ds(s * rps, rps)], degv)
            pltpu.sync_copy(degv, deg_out.at[pl.ds(s * rps, rps)])

    return pl.kernel(
        body,
        out_type=[
            jax.ShapeDtypeStruct((n_pad, d), jnp.float32),
            jax.ShapeDtypeStruct((n_pad,), jnp.float32),
        ],
        mesh=plsc.VectorSubcoreMesh(core_axis_name="c", subcore_axis_name="s"),
        scratch_types=[
            pltpu.VMEM((bpc, CHUNK), jnp.int32),
            pltpu.VMEM((bpc, CHUNK), jnp.int32),
            pltpu.VMEM((CHUNK, d), jnp.float32),
            pltpu.VMEM((CHUNK, d), jnp.float32),
            pltpu.VMEM((CHUNK,), jnp.float32),
            pltpu.VMEM((n_pad // NS,), jnp.float32),
            pltpu.VMEM_SHARED((n_pad, d), jnp.float32),
            pltpu.VMEM_SHARED((n_pad,), jnp.float32),
            pltpu.SemaphoreType.DMA,
            pltpu.SemaphoreType.DMA,
            pltpu.SemaphoreType.DMA,
        ],
    )


def _tc_body(x_ref, a_ref, d_ref, w1t, b1r, w2t, b2r,
             waggt, w3t, b3r, w4t, b4r, o_ref):
    hp = jax.lax.Precision.HIGHEST
    xb = x_ref[...]
    h1 = jnp.maximum(
        jnp.dot(xb, w1t[...], precision=hp,
                preferred_element_type=jnp.float32) + b1r[...], 0.0)
    out1 = jnp.dot(h1, w2t[...], precision=hp,
                   preferred_element_type=jnp.float32) + b2r[...]
    mean = a_ref[...] / jnp.maximum(d_ref[...], 1.0)
    x1 = jnp.dot(mean, waggt[...], precision=hp,
                 preferred_element_type=jnp.float32)
    h2 = jnp.maximum(
        jnp.dot(x1, w3t[...], precision=hp,
                preferred_element_type=jnp.float32) + b3r[...], 0.0)
    out2 = jnp.dot(h2, w4t[...], precision=hp,
                   preferred_element_type=jnp.float32) + b4r[...]
    o_ref[...] = out1 + out2


def kernel(x, edge_index, W1, b1, W2, b2, Wagg, W3, b3, W4, b4):
    n, d = x.shape
    e = edge_index.shape[1]
    d_out = W2.shape[0]
    cps = -(-(-(-e // (NS * CHUNK))) // 8) * 8
    bpc = 40
    while cps % bpc:
        bpc -= 8
    e_pad = cps * CHUNK * NS
    n_pad = -(-(n + 1) // (NS * 16)) * (NS * 16)

    src = edge_index[0]
    dst = edge_index[1]
    fill = jnp.full((e_pad - e,), n, jnp.int32)
    src2d = jnp.concatenate([src, fill]).reshape(e_pad // CHUNK, CHUNK)
    dst2d = jnp.concatenate([dst, fill]).reshape(e_pad // CHUNK, CHUNK)
    x_pad = jnp.concatenate(
        [x, jnp.zeros((n_pad - n, d), jnp.float32)], axis=0)
    zrow = jnp.zeros((n_pad, d), jnp.float32)

    aggf, degf = _make_sc_agg(n_pad, d, cps, bpc)(x_pad, src2d, dst2d, zrow)
    a0 = aggf[:n]
    dg = degf[:n].reshape(n, 1)

    br = next(b for b in (400, 500, 250, 200, 100, 50, 40, 25, 16, 8, 1)
              if n % b == 0)
    grid = (n // br,)
    row_spec = pl.BlockSpec((br, d), lambda i: (i, 0))
    col_spec = pl.BlockSpec((br, 1), lambda i: (i, 0))

    def w_spec(shape):
        return pl.BlockSpec(shape, lambda i: (0,) * len(shape))

    return pl.pallas_call(
        _tc_body,
        grid=grid,
        in_specs=[
            row_spec, row_spec, col_spec,
            w_spec(W1.T.shape), w_spec((1, b1.shape[0])),
            w_spec(W2.T.shape), w_spec((1, b2.shape[0])),
            w_spec(Wagg.T.shape),
            w_spec(W3.T.shape), w_spec((1, b3.shape[0])),
            w_spec(W4.T.shape), w_spec((1, b4.shape[0])),
        ],
        out_specs=pl.BlockSpec((br, d_out), lambda i: (i, 0)),
        out_shape=jax.ShapeDtypeStruct((n, d_out), jnp.float32),
    )(x, a0, dg,
      W1.T, b1.reshape(1, -1), W2.T, b2.reshape(1, -1),
      Wagg.T,
      W3.T, b3.reshape(1, -1), W4.T, b4.reshape(1, -1))


# speedup vs baseline: 1.4964x; 1.4964x over previous
import functools

import jax
import jax.numpy as jnp
from jax import lax
from jax.experimental import pallas as pl
from jax.experimental.pallas import tpu as pltpu
from jax.experimental.pallas import tpu_sc as plsc

NC = 2
NS = 16
NW = NC * NS
CHUNK = 128


def _make_sc_agg(n_pad, d, cpw0, cpw1, bpc):
    rps = n_pad // NS

    def body(x_hbm, src_hbm, dst_hbm, agg_out, deg_out,
             srcv, dstv, rows0, rows1, onesv, degv, agg_sh, deg_sh,
             semg0, semg1, semd):
        c = lax.axis_index("c")
        s = lax.axis_index("s")
        base = jnp.where(c == 0, s * cpw0, NS * cpw0 + s * cpw1)
        nblk = jnp.where(c == 0, cpw0 // bpc, cpw1 // bpc)
        def zrow(i, carry):
            for t in range(d // 16):
                rows0[i, pl.ds(t * 16, 16)] = jnp.zeros((16,), jnp.float32)
            return carry

        lax.fori_loop(0, CHUNK, zrow, 0)
        for q in range(rps // CHUNK):
            pltpu.sync_copy(rows0,
                            agg_sh.at[pl.ds(s * rps + q * CHUNK, CHUNK)])
        if rps % CHUNK:
            pltpu.sync_copy(
                rows0.at[pl.ds(0, rps % CHUNK)],
                agg_sh.at[pl.ds(s * rps + rps // CHUNK * CHUNK,
                                rps % CHUNK)])
        for i in range(rps // 16):
            degv[pl.ds(i * 16, 16)] = jnp.zeros((16,), jnp.float32)
        pltpu.sync_copy(degv, deg_sh.at[pl.ds(s * rps, rps)])
        for i in range(CHUNK // 16):
            onesv[pl.ds(i * 16, 16)] = jnp.ones((16,), jnp.float32)
        plsc.subcore_barrier()

        bufs = (rows0, rows1)
        semgs = (semg0, semg1)

        for k in range(cpw0 // bpc):

            @pl.when(k < nblk)
            def _():
                pltpu.sync_copy(src_hbm.at[pl.ds(base + k * bpc, bpc)],
                                srcv)
                pltpu.sync_copy(dst_hbm.at[pl.ds(base + k * bpc, bpc)],
                                dstv)
                pltpu.async_copy(x_hbm.at[srcv.at[0]], rows0, semg0)
                pltpu.async_copy(x_hbm.at[srcv.at[1]], rows1, semg1)

                def step(i, carry):
                    for b in range(2):
                        j = i * 2 + b
                        rows = bufs[b]
                        pltpu.make_async_copy(x_hbm.at[srcv.at[j]], rows,
                                              semgs[b]).wait()
                        pltpu.async_copy(onesv, deg_sh.at[dstv.at[j]],
                                         semd, add=True)
                        pltpu.sync_copy(rows, agg_sh.at[dstv.at[j]],
                                        add=True)

                        @pl.when(j + 2 < bpc)
                        def _():
                            pltpu.async_copy(x_hbm.at[srcv.at[j + 2]],
                                             rows, semgs[b])
                    return carry

                lax.fori_loop(0, bpc // 2, step, 0)

                def drain(j, carry):
                    pltpu.make_async_copy(onesv, deg_sh.at[dstv.at[j]],
                                          semd).wait()
                    return carry

                lax.fori_loop(0, bpc, drain, 0)

        plsc.subcore_barrier()
        pltpu.sync_copy(agg_sh.at[pl.ds(s * rps, rps)],
                        agg_out.at[pl.ds(c * n_pad + s * rps, rps)])
        pltpu.sync_copy(deg_sh.at[pl.ds(s * rps, rps)], degv)
        pltpu.sync_copy(degv, deg_out.at[pl.ds(c * n_pad + s * rps, rps)])

    return pl.kernel(
        body,
        out_type=[
            jax.ShapeDtypeStruct((2 * n_pad, d), jnp.float32),
            jax.ShapeDtypeStruct((2 * n_pad,), jnp.float32),
        ],
        mesh=plsc.VectorSubcoreMesh(core_axis_name="c", subcore_axis_name="s"),
        scratch_types=[
            pltpu.VMEM((bpc, CHUNK), jnp.int32),
            pltpu.VMEM((bpc, CHUNK), jnp.int32),
            pltpu.VMEM((CHUNK, d), jnp.float32),
            pltpu.VMEM((CHUNK, d), jnp.float32),
            pltpu.VMEM((CHUNK,), jnp.float32),
            pltpu.VMEM((n_pad // NS,), jnp.float32),
            pltpu.VMEM_SHARED((n_pad, d), jnp.float32),
            pltpu.VMEM_SHARED((n_pad,), jnp.float32),
            pltpu.SemaphoreType.DMA,
            pltpu.SemaphoreType.DMA,
            pltpu.SemaphoreType.DMA,
        ],
    )


def _tc_body(x_ref, a0_ref, a1_ref, d0_ref, d1_ref, w1t, b1r, w2t, b2r,
             waggt, w3t, b3r, w4t, b4r, o_ref):
    hp = jax.lax.Precision.HIGHEST
    xb = x_ref[...]
    h1 = jnp.maximum(
        jnp.dot(xb, w1t[...], precision=hp,
                preferred_element_type=jnp.float32) + b1r[...], 0.0)
    out1 = jnp.dot(h1, w2t[...], precision=hp,
                   preferred_element_type=jnp.float32) + b2r[...]
    agg = a0_ref[...] + a1_ref[...]
    deg = d0_ref[...] + d1_ref[...]
    mean = agg / jnp.maximum(deg, 1.0)
    x1 = jnp.dot(mean, waggt[...], precision=hp,
                 preferred_element_type=jnp.float32)
    h2 = jnp.maximum(
        jnp.dot(x1, w3t[...], precision=hp,
                preferred_element_type=jnp.float32) + b3r[...], 0.0)
    out2 = jnp.dot(h2, w4t[...], precision=hp,
                   preferred_element_type=jnp.float32) + b4r[...]
    o_ref[...] = out1 + out2


def kernel(x, edge_index, W1, b1, W2, b2, Wagg, W3, b3, W4, b4):
    n, d = x.shape
    e = edge_index.shape[1]
    d_out = W2.shape[0]
    cpt = -(-(-(-e // (NS * CHUNK))) // 16) * 16
    cpw0 = (3 * cpt // 4) // 8 * 8
    cpw1 = cpt - cpw0
    bpc = 40
    while cpw0 % bpc or cpw1 % bpc:
        bpc -= 8
    e_pad = cpt * CHUNK * NS
    n_pad = -(-(n + 1) // (NS * 16)) * (NS * 16)

    src = edge_index[0]
    dst = edge_index[1]
    fill = jnp.full((e_pad - e,), n, jnp.int32)
    src2d = jnp.concatenate([src, fill]).reshape(e_pad // CHUNK, CHUNK)
    dst2d = jnp.concatenate([dst, fill]).reshape(e_pad // CHUNK, CHUNK)
    x_pad = jnp.concatenate(
        [x, jnp.zeros((n_pad - n, d), jnp.float32)], axis=0)

    aggf, degf = _make_sc_agg(n_pad, d, cpw0, cpw1, bpc)(
        x_pad, src2d, dst2d)
    a0 = aggf[:n]
    a1 = aggf[n_pad:n_pad + n]
    d0 = degf[:n].reshape(n, 1)
    d1 = degf[n_pad:n_pad + n].reshape(n, 1)

    br = next(b for b in (400, 500, 250, 200, 100, 50, 40, 25, 16, 8, 1)
              if n % b == 0)
    grid = (n // br,)
    row_spec = pl.BlockSpec((br, d), lambda i: (i, 0))
    col_spec = pl.BlockSpec((br, 1), lambda i: (i, 0))

    def w_spec(shape):
        return pl.BlockSpec(shape, lambda i: (0,) * len(shape))

    return pl.pallas_call(
        _tc_body,
        grid=grid,
        in_specs=[
            row_spec, row_spec, row_spec, col_spec, col_spec,
            w_spec(W1.T.shape), w_spec((1, b1.shape[0])),
            w_spec(W2.T.shape), w_spec((1, b2.shape[0])),
            w_spec(Wagg.T.shape),
            w_spec(W3.T.shape), w_spec((1, b3.shape[0])),
            w_spec(W4.T.shape), w_spec((1, b4.shape[0])),
        ],
        out_specs=pl.BlockSpec((br, d_out), lambda i: (i, 0)),
        out_shape=jax.ShapeDtypeStruct((n, d_out), jnp.float32),
    )(x, a0, a1, d0, d1,
      W1.T, b1.reshape(1, -1), W2.T, b2.reshape(1, -1),
      Wagg.T,
      W3.T, b3.reshape(1, -1), W4.T, b4.reshape(1, -1))


# speedup vs baseline: 1.5529x; 1.0377x over previous
import functools

import jax
import jax.numpy as jnp
from jax import lax
from jax.experimental import pallas as pl
from jax.experimental.pallas import tpu as pltpu
from jax.experimental.pallas import tpu_sc as plsc

NC = 2
NS = 16
NW = NC * NS
CHUNK = 128


def _make_sc_agg(n_pad, d, cpw0, cpw1, bpc):
    rps = n_pad // NS

    def body(x_hbm, src_hbm, dst_hbm, agg_out, deg_out,
             srcv, dstv, rows0, rows1, onesv, degv, agg_sh, deg_sh,
             semg0, semg1, semd):
        c = lax.axis_index("c")
        s = lax.axis_index("s")
        base = jnp.where(c == 0, s * cpw0, NS * cpw0 + s * cpw1)
        nblk = jnp.where(c == 0, cpw0 // bpc, cpw1 // bpc)
        def zrow(i, carry):
            for t in range(d // 16):
                rows0[i, pl.ds(t * 16, 16)] = jnp.zeros((16,), jnp.float32)
            return carry

        lax.fori_loop(0, CHUNK, zrow, 0)
        for q in range(rps // CHUNK):
            pltpu.sync_copy(rows0,
                            agg_sh.at[pl.ds(s * rps + q * CHUNK, CHUNK)])
        if rps % CHUNK:
            pltpu.sync_copy(
                rows0.at[pl.ds(0, rps % CHUNK)],
                agg_sh.at[pl.ds(s * rps + rps // CHUNK * CHUNK,
                                rps % CHUNK)])
        for i in range(rps // 16):
            degv[pl.ds(i * 16, 16)] = jnp.zeros((16,), jnp.float32)
        pltpu.sync_copy(degv, deg_sh.at[pl.ds(s * rps, rps)])
        for i in range(CHUNK // 16):
            onesv[pl.ds(i * 16, 16)] = jnp.ones((16,), jnp.float32)
        plsc.subcore_barrier()

        bufs = (rows0, rows1)
        semgs = (semg0, semg1)

        for k in range(cpw0 // bpc):

            @pl.when(k < nblk)
            def _():
                pltpu.sync_copy(src_hbm.at[pl.ds(base + k * bpc, bpc)],
                                srcv)
                pltpu.sync_copy(dst_hbm.at[pl.ds(base + k * bpc, bpc)],
                                dstv)
                pltpu.async_copy(x_hbm.at[srcv.at[0]], rows0, semg0)
                pltpu.async_copy(x_hbm.at[srcv.at[1]], rows1, semg1)

                def step(i, carry):
                    for b in range(2):
                        j = i * 2 + b
                        rows = bufs[b]
                        pltpu.make_async_copy(x_hbm.at[srcv.at[j]], rows,
                                              semgs[b]).wait()
                        pltpu.async_copy(onesv, deg_sh.at[dstv.at[j]],
                                         semd, add=True)
                        pltpu.sync_copy(rows, agg_sh.at[dstv.at[j]],
                                        add=True)

                        @pl.when(j + 2 < bpc)
                        def _():
                            pltpu.async_copy(x_hbm.at[srcv.at[j + 2]],
                                             rows, semgs[b])
                    return carry

                lax.fori_loop(0, bpc // 2, step, 0)

                def drain(j, carry):
                    pltpu.make_async_copy(onesv, deg_sh.at[dstv.at[j]],
                                          semd).wait()
                    return carry

                lax.fori_loop(0, bpc, drain, 0)

        plsc.subcore_barrier()
        pltpu.sync_copy(agg_sh.at[pl.ds(s * rps, rps)],
                        agg_out.at[pl.ds(c * n_pad + s * rps, rps)])
        pltpu.sync_copy(deg_sh.at[pl.ds(s * rps, rps)], degv)
        pltpu.sync_copy(degv, deg_out.at[pl.ds(c * n_pad + s * rps, rps)])

    return pl.kernel(
        body,
        out_type=[
            jax.ShapeDtypeStruct((2 * n_pad, d), jnp.float32),
            jax.ShapeDtypeStruct((2 * n_pad,), jnp.float32),
        ],
        mesh=plsc.VectorSubcoreMesh(core_axis_name="c", subcore_axis_name="s"),
        scratch_types=[
            pltpu.VMEM((bpc, CHUNK), jnp.int32),
            pltpu.VMEM((bpc, CHUNK), jnp.int32),
            pltpu.VMEM((CHUNK, d), jnp.float32),
            pltpu.VMEM((CHUNK, d), jnp.float32),
            pltpu.VMEM((CHUNK,), jnp.float32),
            pltpu.VMEM((n_pad // NS,), jnp.float32),
            pltpu.VMEM_SHARED((n_pad, d), jnp.float32),
            pltpu.VMEM_SHARED((n_pad,), jnp.float32),
            pltpu.SemaphoreType.DMA,
            pltpu.SemaphoreType.DMA,
            pltpu.SemaphoreType.DMA,
        ],
    )


def _tc_body(x_ref, a0_ref, a1_ref, d0_ref, d1_ref, w1t, b1r, w2t, b2r,
             waggt, w3t, b3r, w4t, b4r, o_ref):
    hp = jax.lax.Precision.HIGHEST
    xb = x_ref[...]
    h1 = jnp.maximum(
        jnp.dot(xb, w1t[...], precision=hp,
                preferred_element_type=jnp.float32) + b1r[...], 0.0)
    out1 = jnp.dot(h1, w2t[...], precision=hp,
                   preferred_element_type=jnp.float32) + b2r[...]
    agg = a0_ref[...] + a1_ref[...]
    deg = d0_ref[...] + d1_ref[...]
    mean = agg / jnp.maximum(deg, 1.0)
    x1 = jnp.dot(mean, waggt[...], precision=hp,
                 preferred_element_type=jnp.float32)
    h2 = jnp.maximum(
        jnp.dot(x1, w3t[...], precision=hp,
                preferred_element_type=jnp.float32) + b3r[...], 0.0)
    out2 = jnp.dot(h2, w4t[...], precision=hp,
                   preferred_element_type=jnp.float32) + b4r[...]
    o_ref[...] = out1 + out2


def kernel(x, edge_index, W1, b1, W2, b2, Wagg, W3, b3, W4, b4):
    n, d = x.shape
    e = edge_index.shape[1]
    d_out = W2.shape[0]
    cpt = -(-(-(-e // (NS * CHUNK))) // 16) * 16
    cpw0 = (9 * cpt // 10) // 8 * 8
    cpw1 = cpt - cpw0
    bpc = 40
    while cpw0 % bpc or cpw1 % bpc:
        bpc -= 8
    e_pad = cpt * CHUNK * NS
    n_pad = -(-(n + 1) // (NS * 16)) * (NS * 16)

    src = edge_index[0]
    dst = edge_index[1]
    fill = jnp.full((e_pad - e,), n, jnp.int32)
    src2d = jnp.concatenate([src, fill]).reshape(e_pad // CHUNK, CHUNK)
    dst2d = jnp.concatenate([dst, fill]).reshape(e_pad // CHUNK, CHUNK)
    x_pad = jnp.concatenate(
        [x, jnp.zeros((n_pad - n, d), jnp.float32)], axis=0)

    aggf, degf = _make_sc_agg(n_pad, d, cpw0, cpw1, bpc)(
        x_pad, src2d, dst2d)
    a0 = aggf[:n]
    a1 = aggf[n_pad:n_pad + n]
    d0 = degf[:n].reshape(n, 1)
    d1 = degf[n_pad:n_pad + n].reshape(n, 1)

    br = next(b for b in (400, 500, 250, 200, 100, 50, 40, 25, 16, 8, 1)
              if n % b == 0)
    grid = (n // br,)
    row_spec = pl.BlockSpec((br, d), lambda i: (i, 0))
    col_spec = pl.BlockSpec((br, 1), lambda i: (i, 0))

    def w_spec(shape):
        return pl.BlockSpec(shape, lambda i: (0,) * len(shape))

    return pl.pallas_call(
        _tc_body,
        grid=grid,
        in_specs=[
            row_spec, row_spec, row_spec, col_spec, col_spec,
            w_spec(W1.T.shape), w_spec((1, b1.shape[0])),
            w_spec(W2.T.shape), w_spec((1, b2.shape[0])),
            w_spec(Wagg.T.shape),
            w_spec(W3.T.shape), w_spec((1, b3.shape[0])),
            w_spec(W4.T.shape), w_spec((1, b4.shape[0])),
        ],
        out_specs=pl.BlockSpec((br, d_out), lambda i: (i, 0)),
        out_shape=jax.ShapeDtypeStruct((n, d_out), jnp.float32),
    )(x, a0, a1, d0, d1,
      W1.T, b1.reshape(1, -1), W2.T, b2.reshape(1, -1),
      Wagg.T,
      W3.T, b3.reshape(1, -1), W4.T, b4.reshape(1, -1))
